# baseline (speedup 1.0000x reference)
import jax
import jax.numpy as jnp
from jax import lax
from jax.experimental import pallas as pl
from jax.experimental.pallas import tpu as pltpu
from jax.experimental.pallas import tpu_sc as plsc

N_NODES = 10000
N_PAD = 10240
E_EDGES = 320000
E_PAD = 321536
CHUNK = 128
CHUNKS_PER_TILE = E_PAD // (16 * CHUNK)
ACC_ROWS = 10112
STRIPE = ACC_ROWS // 16
DUMMY_DST = N_NODES + 100
ROW_BLK = 512


def _prep_body(x_ref, w_ref, b_ref, wa_ref, xt_ref, s_ref):
    xv = x_ref[...]
    xth = jnp.dot(xv, w_ref[0], preferred_element_type=jnp.float32) + b_ref[0]
    xtk = jnp.dot(xv, w_ref[1], preferred_element_type=jnp.float32) + b_ref[1]
    xt_ref[0] = xth
    xt_ref[1] = xtk
    s_ref[...] = (jnp.dot(xth, wa_ref[0], preferred_element_type=jnp.float32)
                  + jnp.dot(xtk, wa_ref[1], preferred_element_type=jnp.float32))


def _sc_body(xt_hbm, st_hbm, ba_hbm, e2_hbm, out_hbm,
             st_v, rows_v, ed_v, cc_v, ix_v, ba_v, acc, sem_s, sem_e, sem_g):
    c = lax.axis_index("c")
    s = lax.axis_index("s")
    base = s * STRIPE
    gbase = s * 156 + jnp.minimum(s, 4)
    nj = jnp.where(s < 4, 157, 156)

    pltpu.sync_copy(st_hbm.at[c], st_v)
    pltpu.sync_copy(ba_hbm, ba_v)
    bav = plsc.load_gather(ba_v, [jnp.full((16,), c * 16, jnp.int32)])
    cofs = jnp.full((16,), c * N_PAD, jnp.int32)

    def zrow(i, carry):
        for r in range(8):
            rows_v[0, i, pl.ds(r * 16, 16)] = jnp.zeros((16,), jnp.float32)
        return carry
    lax.fori_loop(0, CHUNK, zrow, 0)
    for k5 in range(4):
        pltpu.sync_copy(rows_v.at[0], acc.at[pl.ds(base + k5 * CHUNK, CHUNK)])
    pltpu.sync_copy(rows_v.at[0, pl.ds(0, STRIPE - 4 * CHUNK)],
                    acc.at[pl.ds(base + 4 * CHUNK, STRIPE - 4 * CHUNK)])
    plsc.subcore_barrier()

    pltpu.sync_copy(e2_hbm.at[c, 0, pl.ds(gbase * CHUNK, CHUNK)],
                    ed_v.at[0, 0])
    pltpu.sync_copy(e2_hbm.at[c, 1, pl.ds(gbase * CHUNK, CHUNK)],
                    ed_v.at[0, 1])
    pltpu.async_copy(e2_hbm.at[c, 0, pl.ds((gbase + 1) * CHUNK, CHUNK)],
                     ed_v.at[1, 0], sem_e)
    pltpu.async_copy(e2_hbm.at[c, 1, pl.ds((gbase + 1) * CHUNK, CHUNK)],
                     ed_v.at[1, 1], sem_e)
    pltpu.async_copy(xt_hbm.at[ed_v.at[0, 0]], rows_v.at[0], sem_g)

    def chunk_body(j, carry):
        b = j & 1
        nb = 1 - b

        for i in range(CHUNK // 16):
            sl = pl.ds(i * 16, 16)
            dstv = ed_v[b, 1, sl]
            srcl = ed_v[b, 0, sl] - cofs
            wd = plsc.load_gather(st_v, [dstv >> 7, dstv & 127])
            ws = plsc.load_gather(st_v, [srcl >> 7, srcl & 127])
            z = (plsc.bitcast(wd & jnp.int32(-65536), jnp.float32)
                 + plsc.bitcast(ws << 16, jnp.float32) + bav)
            cc_v[sl] = 1.0 / (1.0 + jnp.exp(-z))
            ix_v[b, sl] = dstv

        @pl.when(j >= 1)
        def _():
            pltpu.make_async_copy(
                xt_hbm.at[pl.ds(0, CHUNK)], rows_v.at[nb], sem_s).wait()

        @pl.when(j < nj - 1)
        def _():
            pltpu.make_async_copy(
                e2_hbm.at[c, 0, pl.ds(0, CHUNK)], ed_v.at[nb, 0], sem_e).wait()
            pltpu.make_async_copy(
                e2_hbm.at[c, 1, pl.ds(0, CHUNK)], ed_v.at[nb, 1], sem_e).wait()
            pltpu.async_copy(xt_hbm.at[ed_v.at[nb, 0]], rows_v.at[nb], sem_g)

        pltpu.make_async_copy(
            xt_hbm.at[pl.ds(0, CHUNK)], rows_v.at[b], sem_g).wait()

        @pl.when(j < nj - 2)
        def _():
            off = (gbase + j + 2) * CHUNK
            pltpu.async_copy(e2_hbm.at[c, 0, pl.ds(off, CHUNK)],
                             ed_v.at[b, 0], sem_e)
            pltpu.async_copy(e2_hbm.at[c, 1, pl.ds(off, CHUNK)],
                             ed_v.at[b, 1], sem_e)

        def edge(e16, cy):
            e0 = e16 * 16
            ces = [cc_v[pl.ds(e0 + u, 16)][0] for u in range(16)]
            for u in range(16):
                for r in range(8):
                    sl = pl.ds(r * 16, 16)
                    rows_v[b, e0 + u, sl] = rows_v[b, e0 + u, sl] * ces[u]
            return cy
        lax.fori_loop(0, CHUNK // 16, edge, 0)

        pltpu.async_copy(rows_v.at[b], acc.at[ix_v.at[b]], sem_s, add=True)
        return carry
    lax.fori_loop(0, nj, chunk_body, 0)
    pltpu.make_async_copy(
        xt_hbm.at[pl.ds(0, CHUNK)], rows_v.at[(nj - 1) & 1], sem_s).wait()
    plsc.subcore_barrier()
    pltpu.sync_copy(acc.at[pl.ds(base, STRIPE)],
                    out_hbm.at[c, pl.ds(base, STRIPE)])


def _gate_body(msg_ref, wg_ref, bg_ref, out_ref):
    h = msg_ref[0]
    k = msg_ref[1]
    logits = (jnp.dot(h, wg_ref[0], preferred_element_type=jnp.float32)
              + jnp.dot(k, wg_ref[1], preferred_element_type=jnp.float32)
              + bg_ref[...])
    g = 1.0 / (1.0 + jnp.exp(-logits))
    out_ref[...] = g[:, 0:1] * h + g[:, 1:2] * k


def kernel(x, hyperedge_index, knn_edge_index,
           W_h, b_h, Wa_h, ba_h,
           W_k, b_k, Wa_k, ba_k,
           Wg, bg):
    f32 = jnp.float32

    xp = jnp.pad(x, ((0, N_PAD - N_NODES), (0, 0)))
    E2 = jnp.stack([hyperedge_index,
                    knn_edge_index + jnp.array([[N_PAD], [0]], jnp.int32)])
    W2 = jnp.stack([W_h, W_k])
    B2 = jnp.stack([b_h, b_k])[:, None, :]
    z128 = jnp.zeros((128,), f32)
    wa0 = jnp.stack([Wa_h[:128, 0], Wa_h[128:, 0], z128, z128], axis=1)
    wa1 = jnp.stack([z128, z128, Wa_k[:128, 0], Wa_k[128:, 0]], axis=1)
    WA = jnp.stack([wa0, wa1])
    BA = jnp.broadcast_to(
        jnp.concatenate([ba_h, ba_k])[:, None], (2, 16)).astype(f32).reshape(32)

    grid = N_PAD // ROW_BLK
    xt, scores = pl.pallas_call(
        _prep_body,
        grid=(grid,),
        in_specs=[
            pl.BlockSpec((ROW_BLK, 128), lambda i: (i, 0)),
            pl.BlockSpec((2, 128, 128), lambda i: (0, 0, 0)),
            pl.BlockSpec((2, 1, 128), lambda i: (0, 0, 0)),
            pl.BlockSpec((2, 128, 4), lambda i: (0, 0, 0)),
        ],
        out_specs=[
            pl.BlockSpec((2, ROW_BLK, 128), lambda i: (0, i, 0)),
            pl.BlockSpec((ROW_BLK, 4), lambda i: (i, 0)),
        ],
        out_shape=[
            jax.ShapeDtypeStruct((2, N_PAD, 128), f32),
            jax.ShapeDtypeStruct((N_PAD, 4), f32),
        ],
    )(xp, W2, B2, WA)

    xt2 = xt.reshape(2 * N_PAD, 128)
    sd_all = jnp.stack([scores[:, 0], scores[:, 2]])
    ss_all = jnp.stack([scores[:, 1], scores[:, 3]])
    u_sd = lax.bitcast_convert_type(
        sd_all.astype(jnp.bfloat16), jnp.uint16).astype(jnp.uint32)
    u_ss = lax.bitcast_convert_type(
        ss_all.astype(jnp.bfloat16), jnp.uint16).astype(jnp.uint32)
    st32 = lax.bitcast_convert_type(
        (u_sd << 16) | u_ss, jnp.int32).reshape(2, N_PAD // 128, 128)

    mesh = plsc.VectorSubcoreMesh(core_axis_name="c", subcore_axis_name="s")
    msg = pl.kernel(
        _sc_body,
        out_type=jax.ShapeDtypeStruct((2, N_PAD, 128), f32),
        mesh=mesh,
        compiler_params=pltpu.CompilerParams(needs_layout_passes=False),
        scratch_types=[
            pltpu.VMEM((N_PAD // 128, 128), jnp.int32),
            pltpu.VMEM((2, CHUNK, 128), f32),
            pltpu.VMEM((2, 2, CHUNK), jnp.int32),
            pltpu.VMEM((CHUNK + 16,), f32),
            pltpu.VMEM((2, CHUNK), jnp.int32),
            pltpu.VMEM((32,), f32),
            pltpu.VMEM_SHARED((ACC_ROWS, 128), f32),
            pltpu.SemaphoreType.DMA,
            pltpu.SemaphoreType.DMA,
            pltpu.SemaphoreType.DMA,
        ],
    )(xt2, st32, BA, E2)

    WG = jnp.stack([Wg[:128], Wg[128:]])
    BG = bg[None, :]
    gated = pl.pallas_call(
        _gate_body,
        grid=(grid,),
        in_specs=[
            pl.BlockSpec((2, ROW_BLK, 128), lambda i: (0, i, 0)),
            pl.BlockSpec((2, 128, 2), lambda i: (0, 0, 0)),
            pl.BlockSpec((1, 2), lambda i: (0, 0)),
        ],
        out_specs=pl.BlockSpec((ROW_BLK, 128), lambda i: (i, 0)),
        out_shape=jax.ShapeDtypeStruct((N_NODES, 128), f32),
    )(msg, WG, BG)

    return gated


# speedup vs baseline: 2.8587x; 2.8587x over previous
import jax
import jax.numpy as jnp
from jax import lax
from jax.experimental import pallas as pl
from jax.experimental.pallas import tpu as pltpu
from jax.experimental.pallas import tpu_sc as plsc

N_NODES = 10000
N_PAD = 10240
E_EDGES = 320000
E_PAD = 321536
CHUNK = 128
CHUNKS_PER_TILE = E_PAD // (16 * CHUNK)
ACC_ROWS = 10112
STRIPE = ACC_ROWS // 16
DUMMY_DST = N_NODES + 100
ROW_BLK = 512


def _prep_body(x_ref, w_ref, b_ref, wa_ref, xt_ref, s_ref):
    xv = x_ref[...]
    xth = jnp.dot(xv, w_ref[0], preferred_element_type=jnp.float32) + b_ref[0]
    xtk = jnp.dot(xv, w_ref[1], preferred_element_type=jnp.float32) + b_ref[1]
    xt_ref[0] = xth
    xt_ref[1] = xtk
    s_ref[...] = (jnp.dot(xth, wa_ref[0], preferred_element_type=jnp.float32)
                  + jnp.dot(xtk, wa_ref[1], preferred_element_type=jnp.float32))


def _sc_body(xt_hbm, st_hbm, ba_hbm, e2_hbm, out_hbm,
             st_v, rows_v, ed_v, cc_v, ix_v, ba_v, acc, sem_s, sem_e, sem_g):
    c = lax.axis_index("c")
    s = lax.axis_index("s")
    base = s * STRIPE
    gbase = s * 156 + jnp.minimum(s, 4)
    nj = jnp.where(s < 4, 157, 156)

    pltpu.sync_copy(st_hbm.at[c], st_v)
    pltpu.sync_copy(ba_hbm, ba_v)
    bav = plsc.load_gather(ba_v, [jnp.full((16,), c * 16, jnp.int32)])
    cofs = jnp.full((16,), c * N_PAD, jnp.int32)

    def zrow(i, carry):
        for r in range(8):
            rows_v[0, i, pl.ds(r * 16, 16)] = jnp.zeros((16,), jnp.float32)
        return carry
    lax.fori_loop(0, CHUNK, zrow, 0)
    for k5 in range(4):
        pltpu.sync_copy(rows_v.at[0], acc.at[pl.ds(base + k5 * CHUNK, CHUNK)])
    pltpu.sync_copy(rows_v.at[0, pl.ds(0, STRIPE - 4 * CHUNK)],
                    acc.at[pl.ds(base + 4 * CHUNK, STRIPE - 4 * CHUNK)])
    plsc.subcore_barrier()

    pltpu.sync_copy(e2_hbm.at[c, 0, pl.ds(gbase * CHUNK, CHUNK)],
                    ed_v.at[0, 0])
    pltpu.sync_copy(e2_hbm.at[c, 1, pl.ds(gbase * CHUNK, CHUNK)],
                    ed_v.at[0, 1])
    pltpu.async_copy(e2_hbm.at[c, 0, pl.ds((gbase + 1) * CHUNK, CHUNK)],
                     ed_v.at[1, 0], sem_e)
    pltpu.async_copy(e2_hbm.at[c, 1, pl.ds((gbase + 1) * CHUNK, CHUNK)],
                     ed_v.at[1, 1], sem_e)
    pltpu.async_copy(xt_hbm.at[ed_v.at[0, 0]], rows_v.at[0], sem_g)

    def chunk_body(j, carry):
        b = j & 1
        nb = 1 - b

        for i in range(CHUNK // 16):
            sl = pl.ds(i * 16, 16)
            dstv = ed_v[b, 1, sl]
            srcl = ed_v[b, 0, sl] - cofs
            wd = plsc.load_gather(st_v, [dstv >> 7, dstv & 127])
            ws = plsc.load_gather(st_v, [srcl >> 7, srcl & 127])
            z = (plsc.bitcast(wd & jnp.int32(-65536), jnp.float32)
                 + plsc.bitcast(ws << 16, jnp.float32) + bav)
            cc_v[sl] = 1.0 / (1.0 + jnp.exp(-z))
            ix_v[b, sl] = dstv

        @pl.when(j >= 1)
        def _():
            pltpu.make_async_copy(
                xt_hbm.at[pl.ds(0, CHUNK)], rows_v.at[nb], sem_s).wait()

        @pl.when(j < nj - 1)
        def _():
            pltpu.make_async_copy(
                e2_hbm.at[c, 0, pl.ds(0, CHUNK)], ed_v.at[nb, 0], sem_e).wait()
            pltpu.make_async_copy(
                e2_hbm.at[c, 1, pl.ds(0, CHUNK)], ed_v.at[nb, 1], sem_e).wait()
            pltpu.async_copy(xt_hbm.at[ed_v.at[nb, 0]], rows_v.at[nb], sem_g)

        pltpu.make_async_copy(
            xt_hbm.at[pl.ds(0, CHUNK)], rows_v.at[b], sem_g).wait()

        @pl.when(j < nj - 2)
        def _():
            off = (gbase + j + 2) * CHUNK
            pltpu.async_copy(e2_hbm.at[c, 0, pl.ds(off, CHUNK)],
                             ed_v.at[b, 0], sem_e)
            pltpu.async_copy(e2_hbm.at[c, 1, pl.ds(off, CHUNK)],
                             ed_v.at[b, 1], sem_e)

        def edge(e8, cy):
            e0 = e8 * 8
            ces = [cc_v[pl.ds(e0 + u, 16)][0] for u in range(8)]
            for u in range(8):
                for r in range(8):
                    sl = pl.ds(r * 16, 16)
                    rows_v[b, e0 + u, sl] = rows_v[b, e0 + u, sl] * ces[u]
            return cy
        lax.fori_loop(0, CHUNK // 8, edge, 0)

        pltpu.async_copy(rows_v.at[b], acc.at[ix_v.at[b]], sem_s, add=True)
        return carry
    lax.fori_loop(0, nj, chunk_body, 0)
    pltpu.make_async_copy(
        xt_hbm.at[pl.ds(0, CHUNK)], rows_v.at[(nj - 1) & 1], sem_s).wait()
    plsc.subcore_barrier()
    pltpu.sync_copy(acc.at[pl.ds(base, STRIPE)],
                    out_hbm.at[c, pl.ds(base, STRIPE)])


def _gate_body(msg_ref, wg_ref, bg_ref, out_ref):
    h = msg_ref[0]
    k = msg_ref[1]
    logits = (jnp.dot(h, wg_ref[0], preferred_element_type=jnp.float32)
              + jnp.dot(k, wg_ref[1], preferred_element_type=jnp.float32)
              + bg_ref[...])
    g = 1.0 / (1.0 + jnp.exp(-logits))
    out_ref[...] = g[:, 0:1] * h + g[:, 1:2] * k


def kernel(x, hyperedge_index, knn_edge_index,
           W_h, b_h, Wa_h, ba_h,
           W_k, b_k, Wa_k, ba_k,
           Wg, bg):
    f32 = jnp.float32

    xp = jnp.pad(x, ((0, N_PAD - N_NODES), (0, 0)))
    E2 = jnp.stack([hyperedge_index,
                    knn_edge_index + jnp.array([[N_PAD], [0]], jnp.int32)])
    W2 = jnp.stack([W_h, W_k])
    B2 = jnp.stack([b_h, b_k])[:, None, :]
    z128 = jnp.zeros((128,), f32)
    wa0 = jnp.stack([Wa_h[:128, 0], Wa_h[128:, 0], z128, z128], axis=1)
    wa1 = jnp.stack([z128, z128, Wa_k[:128, 0], Wa_k[128:, 0]], axis=1)
    WA = jnp.stack([wa0, wa1])
    BA = jnp.broadcast_to(
        jnp.concatenate([ba_h, ba_k])[:, None], (2, 16)).astype(f32).reshape(32)

    grid = N_PAD // ROW_BLK
    xt, scores = pl.pallas_call(
        _prep_body,
        grid=(grid,),
        in_specs=[
            pl.BlockSpec((ROW_BLK, 128), lambda i: (i, 0)),
            pl.BlockSpec((2, 128, 128), lambda i: (0, 0, 0)),
            pl.BlockSpec((2, 1, 128), lambda i: (0, 0, 0)),
            pl.BlockSpec((2, 128, 4), lambda i: (0, 0, 0)),
        ],
        out_specs=[
            pl.BlockSpec((2, ROW_BLK, 128), lambda i: (0, i, 0)),
            pl.BlockSpec((ROW_BLK, 4), lambda i: (i, 0)),
        ],
        out_shape=[
            jax.ShapeDtypeStruct((2, N_PAD, 128), f32),
            jax.ShapeDtypeStruct((N_PAD, 4), f32),
        ],
    )(xp, W2, B2, WA)

    xt2 = xt.reshape(2 * N_PAD, 128)
    sd_all = jnp.stack([scores[:, 0], scores[:, 2]])
    ss_all = jnp.stack([scores[:, 1], scores[:, 3]])
    u_sd = lax.bitcast_convert_type(
        sd_all.astype(jnp.bfloat16), jnp.uint16).astype(jnp.uint32)
    u_ss = lax.bitcast_convert_type(
        ss_all.astype(jnp.bfloat16), jnp.uint16).astype(jnp.uint32)
    st32 = lax.bitcast_convert_type(
        (u_sd << 16) | u_ss, jnp.int32).reshape(2, N_PAD // 128, 128)

    mesh = plsc.VectorSubcoreMesh(core_axis_name="c", subcore_axis_name="s")
    msg = pl.kernel(
        _sc_body,
        out_type=jax.ShapeDtypeStruct((2, N_PAD, 128), f32),
        mesh=mesh,
        compiler_params=pltpu.CompilerParams(needs_layout_passes=False),
        scratch_types=[
            pltpu.VMEM((N_PAD // 128, 128), jnp.int32),
            pltpu.VMEM((2, CHUNK, 128), f32),
            pltpu.VMEM((2, 2, CHUNK), jnp.int32),
            pltpu.VMEM((CHUNK + 16,), f32),
            pltpu.VMEM((2, CHUNK), jnp.int32),
            pltpu.VMEM((32,), f32),
            pltpu.VMEM_SHARED((ACC_ROWS, 128), f32),
            pltpu.SemaphoreType.DMA,
            pltpu.SemaphoreType.DMA,
            pltpu.SemaphoreType.DMA,
        ],
    )(xt2, st32, BA, E2)

    WG = jnp.stack([Wg[:128], Wg[128:]])
    BG = bg[None, :]
    gated = pl.pallas_call(
        _gate_body,
        grid=(grid,),
        in_specs=[
            pl.BlockSpec((2, ROW_BLK, 128), lambda i: (0, i, 0)),
            pl.BlockSpec((2, 128, 2), lambda i: (0, 0, 0)),
            pl.BlockSpec((1, 2), lambda i: (0, 0)),
        ],
        out_specs=pl.BlockSpec((ROW_BLK, 128), lambda i: (i, 0)),
        out_shape=jax.ShapeDtypeStruct((N_NODES, 128), f32),
    )(msg, WG, BG)

    return gated
